# baseline (speedup 1.0000x reference)
import functools

import jax
import jax.numpy as jnp
from jax import lax
from jax.experimental import pallas as pl
from jax.experimental.pallas import tpu as pltpu
from jax.experimental.pallas import tpu_sc as plsc

_N = 10000
_D = 128
_E = 320000

_NUM_CORES = 2
_NUM_SUBCORES = 16
_NW = _NUM_CORES * _NUM_SUBCORES
_CHUNK = 64
_NCHUNK = 160
_EPAD = _NW * _NCHUNK * _CHUNK
_NPAD = 10240
_ROWS_PER_TILE = _NPAD // _NUM_SUBCORES
_ZROWS = 128

_ROW_BLOCK = 1000
_GRID = _N // _ROW_BLOCK


def _make_seg_sum():
  mesh = plsc.VectorSubcoreMesh(core_axis_name="c", subcore_axis_name="s")

  @functools.partial(
      pl.kernel,
      mesh=mesh,
      out_type=jax.ShapeDtypeStruct((_NUM_CORES, _NPAD, _D), jnp.float32),
      scratch_types=[
          pltpu.VMEM((_CHUNK,), jnp.int32),
          pltpu.VMEM((_CHUNK,), jnp.int32),
          pltpu.VMEM((_CHUNK, _D), jnp.float32),
          pltpu.VMEM((_ZROWS, _D), jnp.float32),
          pltpu.VMEM_SHARED((_NPAD, _D), jnp.float32),
          pltpu.SemaphoreType.DMA,
      ],
  )
  def seg_sum(table_hbm, u_hbm, v_hbm, out_hbm, u_v, v_v, rows_v, stage_v,
              acc_sh, sem):
    c = lax.axis_index("c")
    s = lax.axis_index("s")
    wid = s * _NUM_CORES + c
    base0 = wid * _NCHUNK * _CHUNK

    def zero_row(r, _):
      def zero_col(cc, _):
        stage_v[r, pl.ds(cc * 16, 16)] = jnp.zeros((16,), jnp.float32)
        return 0
      return lax.fori_loop(0, _D // 16, zero_col, 0)
    lax.fori_loop(0, _ZROWS, zero_row, 0)

    row0 = s * _ROWS_PER_TILE
    def zcopy(k, _):
      pltpu.sync_copy(stage_v, acc_sh.at[pl.ds(row0 + k * _ZROWS, _ZROWS)])
      return 0
    lax.fori_loop(0, _ROWS_PER_TILE // _ZROWS, zcopy, 0)

    plsc.subcore_barrier()

    def body(j, _):
      base = pl.multiple_of(base0 + j * _CHUNK, 8)
      pltpu.sync_copy(u_hbm.at[pl.ds(base, _CHUNK)], u_v)
      pltpu.sync_copy(v_hbm.at[pl.ds(base, _CHUNK)], v_v)
      pltpu.async_copy(table_hbm.at[u_v], rows_v, sem).wait()
      pltpu.sync_copy(rows_v, acc_sh.at[v_v], add=True)
      return 0
    lax.fori_loop(0, _NCHUNK, body, 0)

    plsc.subcore_barrier()

    def wb(k, _):
      sl = pl.ds(row0 + k * _ZROWS, _ZROWS)
      pltpu.sync_copy(acc_sh.at[sl], stage_v)
      pltpu.sync_copy(stage_v, out_hbm.at[c].at[sl])
      return 0
    lax.fori_loop(0, _ROWS_PER_TILE // _ZROWS, wb, 0)

  return seg_sum


_seg_sum = _make_seg_sum()


def _dotT(a, w):
  return lax.dot_general(a, w, (((1,), (1,)), ((), ())),
                         preferred_element_type=jnp.float32)


def _proj_body(x_ref, w_ref, y_ref, ge_ref):
  x = x_ref[...]
  y_ref[...] = _dotT(x, w_ref[...])
  @pl.when(pl.program_id(0) == 0)
  def _():
    ge_ref[...] = jnp.zeros_like(ge_ref)
  ge_ref[...] += jnp.sum(x, axis=0, keepdims=True)


def _stage_mid_body(y_ref, p0_ref, p1_ref, b1_ref, w2_ref, b2_ref, wn_ref,
                    ynext_ref, ge_ref):
  z = jnp.maximum(y_ref[...] + p0_ref[0] + p1_ref[0] + b1_ref[...], 0.0)
  ne = jnp.maximum(_dotT(z, w2_ref[...]) + b2_ref[...], 0.0)
  ynext_ref[...] = _dotT(ne, wn_ref[...])
  @pl.when(pl.program_id(0) == 0)
  def _():
    ge_ref[...] = jnp.zeros_like(ge_ref)
  ge_ref[...] += jnp.sum(ne, axis=0, keepdims=True)


def _stage_last_body(y_ref, p0_ref, p1_ref, b1_ref, w2_ref, b2_ref,
                     ne_ref, ge_ref):
  z = jnp.maximum(y_ref[...] + p0_ref[0] + p1_ref[0] + b1_ref[...], 0.0)
  ne = jnp.maximum(_dotT(z, w2_ref[...]) + b2_ref[...], 0.0)
  ne_ref[...] = ne
  @pl.when(pl.program_id(0) == 0)
  def _():
    ge_ref[...] = jnp.zeros_like(ge_ref)
  ge_ref[...] += jnp.sum(ne, axis=0, keepdims=True)


def _final_body(ne_ref, ge0_ref, ge1_ref, ge2_ref, g0w_ref, g0b_ref,
                g1w_ref, g1b_ref, g2w_ref, g2b_ref, glw_ref, glb_ref,
                out_ref):
  o0 = jnp.maximum(_dotT(ge0_ref[...], g0w_ref[...]) + g0b_ref[...], 0.0)
  o1 = jnp.maximum(_dotT(ge1_ref[...], g1w_ref[...]) + g1b_ref[...], 0.0)
  o2 = jnp.maximum(_dotT(ge2_ref[...], g2w_ref[...]) + g2b_ref[...], 0.0)
  s = o0 + o1 + o2
  olast = jnp.maximum(_dotT(s, glw_ref[...]) + glb_ref[...], 0.0)
  out_ref[...] = ne_ref[...] + olast


def _row_spec(d):
  return pl.BlockSpec((_ROW_BLOCK, d), lambda i: (i, 0))


def _full_spec(shape):
  return pl.BlockSpec(shape, lambda i: tuple(0 for _ in shape))


def _proj(x, w, d_in):
  return pl.pallas_call(
      _proj_body,
      grid=(_GRID,),
      in_specs=[_row_spec(d_in), _full_spec(w.shape)],
      out_specs=[_row_spec(_D), _full_spec((1, d_in))],
      out_shape=[jax.ShapeDtypeStruct((_N, _D), jnp.float32),
                 jax.ShapeDtypeStruct((1, d_in), jnp.float32)],
  )(x, w)


def _stage_mid(y, p, b1, w2, b2, wn):
  return pl.pallas_call(
      _stage_mid_body,
      grid=(_GRID,),
      in_specs=[_row_spec(_D),
                pl.BlockSpec((1, _ROW_BLOCK, _D), lambda i: (0, i, 0)),
                pl.BlockSpec((1, _ROW_BLOCK, _D), lambda i: (1, i, 0)),
                _full_spec((1, _D)), _full_spec((_D, _D)),
                _full_spec((1, _D)), _full_spec((_D, _D))],
      out_specs=[_row_spec(_D), _full_spec((1, _D))],
      out_shape=[jax.ShapeDtypeStruct((_N, _D), jnp.float32),
                 jax.ShapeDtypeStruct((1, _D), jnp.float32)],
  )(y, p, p, b1, w2, b2, wn)


def _stage_last(y, p, b1, w2, b2):
  return pl.pallas_call(
      _stage_last_body,
      grid=(_GRID,),
      in_specs=[_row_spec(_D),
                pl.BlockSpec((1, _ROW_BLOCK, _D), lambda i: (0, i, 0)),
                pl.BlockSpec((1, _ROW_BLOCK, _D), lambda i: (1, i, 0)),
                _full_spec((1, _D)), _full_spec((_D, _D)),
                _full_spec((1, _D))],
      out_specs=[_row_spec(_D), _full_spec((1, _D))],
      out_shape=[jax.ShapeDtypeStruct((_N, _D), jnp.float32),
                 jax.ShapeDtypeStruct((1, _D), jnp.float32)],
  )(y, p, p, b1, w2, b2)


def _final(ne2, ge0, ge1, ge2, g0w, g0b, g1w, g1b, g2w, g2b, glw, glb):
  d_in = ge0.shape[1]
  return pl.pallas_call(
      _final_body,
      grid=(_GRID,),
      in_specs=[_row_spec(_D),
                _full_spec((1, d_in)), _full_spec((1, _D)), _full_spec((1, _D)),
                _full_spec((_D, d_in)), _full_spec((1, _D)),
                _full_spec((_D, _D)), _full_spec((1, _D)),
                _full_spec((_D, _D)), _full_spec((1, _D)),
                _full_spec((_D, _D)), _full_spec((1, _D))],
      out_specs=_row_spec(_D),
      out_shape=jax.ShapeDtypeStruct((_N, _D), jnp.float32),
  )(ne2, ge0, ge1, ge2, g0w, g0b, g1w, g1b, g2w, g2b, glw, glb)


@jax.jit
def _run(x, u, v, g0_W, g0_b, g1_W, g1_b, g2_W, g2_b, gl_W, gl_b,
         m1_W1, m1_b1, m1_W2, m1_b2, m2_W1, m2_b1, m2_W2, m2_b2):
  r = lambda b: b.reshape(1, -1)

  y1, ge0 = _proj(x, m1_W1, x.shape[1])

  p1 = _seg_sum(y1, u, v)
  y2, ge1 = _stage_mid(y1, p1, r(m1_b1), m1_W2, r(m1_b2), m2_W1)

  p2 = _seg_sum(y2, u, v)
  ne2, ge2 = _stage_last(y2, p2, r(m2_b1), m2_W2, r(m2_b2))

  return _final(ne2, ge0, ge1, ge2, g0_W, r(g0_b), g1_W, r(g1_b),
                g2_W, r(g2_b), gl_W, r(gl_b))


def kernel(x, edge_index, g0_W, g0_b, g1_W, g1_b, g2_W, g2_b, gl_W, gl_b,
           m1_W1, m1_b1, m1_W2, m1_b2, m2_W1, m2_b1, m2_W2, m2_b2):
  npad_e = _EPAD - _E
  if npad_e:
    u = jnp.concatenate([edge_index[0], jnp.zeros((npad_e,), jnp.int32)])
    v = jnp.concatenate(
        [edge_index[1], jnp.full((npad_e,), _NPAD - 1, jnp.int32)])
  else:
    u = edge_index[0]
    v = edge_index[1]
  return _run(x, u, v, g0_W, g0_b, g1_W, g1_b, g2_W, g2_b, gl_W, gl_b,
              m1_W1, m1_b1, m1_W2, m1_b2, m2_W1, m2_b1, m2_W2, m2_b2)


# speedup vs baseline: 1.0573x; 1.0573x over previous
import functools

import jax
import jax.numpy as jnp
from jax import lax
from jax.experimental import pallas as pl
from jax.experimental.pallas import tpu as pltpu
from jax.experimental.pallas import tpu_sc as plsc

_N = 10000
_D = 128
_E = 320000

_NUM_CORES = 2
_NUM_SUBCORES = 16
_NW = _NUM_CORES * _NUM_SUBCORES
_CHUNK = 80
_NCHUNK = 128
_EPAD = _NW * _NCHUNK * _CHUNK
_NPAD = 10240
_ROWS_PER_TILE = _NPAD // _NUM_SUBCORES
_ZROWS = 128

_ROW_BLOCK = 1000
_GRID = _N // _ROW_BLOCK


def _make_seg_sum():
  mesh = plsc.VectorSubcoreMesh(core_axis_name="c", subcore_axis_name="s")

  @functools.partial(
      pl.kernel,
      mesh=mesh,
      out_type=jax.ShapeDtypeStruct((_NUM_CORES, _NPAD, _D), jnp.float32),
      scratch_types=[
          pltpu.VMEM((_CHUNK,), jnp.int32),
          pltpu.VMEM((_CHUNK,), jnp.int32),
          pltpu.VMEM((_CHUNK, _D), jnp.float32),
          pltpu.VMEM((_ZROWS, _D), jnp.float32),
          pltpu.VMEM_SHARED((_NPAD, _D), jnp.float32),
          pltpu.SemaphoreType.DMA,
      ],
  )
  def seg_sum(table_hbm, u_hbm, v_hbm, out_hbm, u_v, v_v, rows_v, stage_v,
              acc_sh, sem):
    c = lax.axis_index("c")
    s = lax.axis_index("s")
    wid = s * _NUM_CORES + c
    base0 = wid * _NCHUNK * _CHUNK

    def zero_row(r, _):
      def zero_col(cc, _):
        stage_v[r, pl.ds(cc * 16, 16)] = jnp.zeros((16,), jnp.float32)
        return 0
      return lax.fori_loop(0, _D // 16, zero_col, 0)
    lax.fori_loop(0, _ZROWS, zero_row, 0)

    row0 = s * _ROWS_PER_TILE
    def zcopy(k, _):
      pltpu.sync_copy(stage_v, acc_sh.at[pl.ds(row0 + k * _ZROWS, _ZROWS)])
      return 0
    lax.fori_loop(0, _ROWS_PER_TILE // _ZROWS, zcopy, 0)

    plsc.subcore_barrier()

    def body(j, _):
      base = pl.multiple_of(base0 + j * _CHUNK, 8)
      pltpu.sync_copy(u_hbm.at[pl.ds(base, _CHUNK)], u_v)
      pltpu.sync_copy(v_hbm.at[pl.ds(base, _CHUNK)], v_v)
      pltpu.async_copy(table_hbm.at[u_v], rows_v, sem).wait()
      pltpu.sync_copy(rows_v, acc_sh.at[v_v], add=True)
      return 0
    lax.fori_loop(0, _NCHUNK, body, 0)

    plsc.subcore_barrier()

    def wb(k, _):
      sl = pl.ds(row0 + k * _ZROWS, _ZROWS)
      pltpu.sync_copy(acc_sh.at[sl], stage_v)
      pltpu.sync_copy(stage_v, out_hbm.at[c].at[sl])
      return 0
    lax.fori_loop(0, _ROWS_PER_TILE // _ZROWS, wb, 0)

  return seg_sum


_seg_sum = _make_seg_sum()


def _dotT(a, w):
  return lax.dot_general(a, w, (((1,), (1,)), ((), ())),
                         preferred_element_type=jnp.float32)


def _proj_body(x_ref, w_ref, y_ref, ge_ref):
  x = x_ref[...]
  y_ref[...] = _dotT(x, w_ref[...])
  @pl.when(pl.program_id(0) == 0)
  def _():
    ge_ref[...] = jnp.zeros_like(ge_ref)
  ge_ref[...] += jnp.sum(x, axis=0, keepdims=True)


def _stage_mid_body(y_ref, p0_ref, p1_ref, b1_ref, w2_ref, b2_ref, wn_ref,
                    ynext_ref, ge_ref):
  z = jnp.maximum(y_ref[...] + p0_ref[0] + p1_ref[0] + b1_ref[...], 0.0)
  ne = jnp.maximum(_dotT(z, w2_ref[...]) + b2_ref[...], 0.0)
  ynext_ref[...] = _dotT(ne, wn_ref[...])
  @pl.when(pl.program_id(0) == 0)
  def _():
    ge_ref[...] = jnp.zeros_like(ge_ref)
  ge_ref[...] += jnp.sum(ne, axis=0, keepdims=True)


def _stage_last_body(y_ref, p0_ref, p1_ref, b1_ref, w2_ref, b2_ref,
                     ne_ref, ge_ref):
  z = jnp.maximum(y_ref[...] + p0_ref[0] + p1_ref[0] + b1_ref[...], 0.0)
  ne = jnp.maximum(_dotT(z, w2_ref[...]) + b2_ref[...], 0.0)
  ne_ref[...] = ne
  @pl.when(pl.program_id(0) == 0)
  def _():
    ge_ref[...] = jnp.zeros_like(ge_ref)
  ge_ref[...] += jnp.sum(ne, axis=0, keepdims=True)


def _final_body(ne_ref, ge0_ref, ge1_ref, ge2_ref, g0w_ref, g0b_ref,
                g1w_ref, g1b_ref, g2w_ref, g2b_ref, glw_ref, glb_ref,
                out_ref):
  o0 = jnp.maximum(_dotT(ge0_ref[...], g0w_ref[...]) + g0b_ref[...], 0.0)
  o1 = jnp.maximum(_dotT(ge1_ref[...], g1w_ref[...]) + g1b_ref[...], 0.0)
  o2 = jnp.maximum(_dotT(ge2_ref[...], g2w_ref[...]) + g2b_ref[...], 0.0)
  s = o0 + o1 + o2
  olast = jnp.maximum(_dotT(s, glw_ref[...]) + glb_ref[...], 0.0)
  out_ref[...] = ne_ref[...] + olast


def _row_spec(d):
  return pl.BlockSpec((_ROW_BLOCK, d), lambda i: (i, 0))


def _full_spec(shape):
  return pl.BlockSpec(shape, lambda i: tuple(0 for _ in shape))


def _proj(x, w, d_in):
  return pl.pallas_call(
      _proj_body,
      grid=(_GRID,),
      in_specs=[_row_spec(d_in), _full_spec(w.shape)],
      out_specs=[_row_spec(_D), _full_spec((1, d_in))],
      out_shape=[jax.ShapeDtypeStruct((_N, _D), jnp.float32),
                 jax.ShapeDtypeStruct((1, d_in), jnp.float32)],
  )(x, w)


def _stage_mid(y, p, b1, w2, b2, wn):
  return pl.pallas_call(
      _stage_mid_body,
      grid=(_GRID,),
      in_specs=[_row_spec(_D),
                pl.BlockSpec((1, _ROW_BLOCK, _D), lambda i: (0, i, 0)),
                pl.BlockSpec((1, _ROW_BLOCK, _D), lambda i: (1, i, 0)),
                _full_spec((1, _D)), _full_spec((_D, _D)),
                _full_spec((1, _D)), _full_spec((_D, _D))],
      out_specs=[_row_spec(_D), _full_spec((1, _D))],
      out_shape=[jax.ShapeDtypeStruct((_N, _D), jnp.float32),
                 jax.ShapeDtypeStruct((1, _D), jnp.float32)],
  )(y, p, p, b1, w2, b2, wn)


def _stage_last(y, p, b1, w2, b2):
  return pl.pallas_call(
      _stage_last_body,
      grid=(_GRID,),
      in_specs=[_row_spec(_D),
                pl.BlockSpec((1, _ROW_BLOCK, _D), lambda i: (0, i, 0)),
                pl.BlockSpec((1, _ROW_BLOCK, _D), lambda i: (1, i, 0)),
                _full_spec((1, _D)), _full_spec((_D, _D)),
                _full_spec((1, _D))],
      out_specs=[_row_spec(_D), _full_spec((1, _D))],
      out_shape=[jax.ShapeDtypeStruct((_N, _D), jnp.float32),
                 jax.ShapeDtypeStruct((1, _D), jnp.float32)],
  )(y, p, p, b1, w2, b2)


def _final(ne2, ge0, ge1, ge2, g0w, g0b, g1w, g1b, g2w, g2b, glw, glb):
  d_in = ge0.shape[1]
  return pl.pallas_call(
      _final_body,
      grid=(_GRID,),
      in_specs=[_row_spec(_D),
                _full_spec((1, d_in)), _full_spec((1, _D)), _full_spec((1, _D)),
                _full_spec((_D, d_in)), _full_spec((1, _D)),
                _full_spec((_D, _D)), _full_spec((1, _D)),
                _full_spec((_D, _D)), _full_spec((1, _D)),
                _full_spec((_D, _D)), _full_spec((1, _D))],
      out_specs=_row_spec(_D),
      out_shape=jax.ShapeDtypeStruct((_N, _D), jnp.float32),
  )(ne2, ge0, ge1, ge2, g0w, g0b, g1w, g1b, g2w, g2b, glw, glb)


@jax.jit
def _run(x, u, v, g0_W, g0_b, g1_W, g1_b, g2_W, g2_b, gl_W, gl_b,
         m1_W1, m1_b1, m1_W2, m1_b2, m2_W1, m2_b1, m2_W2, m2_b2):
  r = lambda b: b.reshape(1, -1)

  y1, ge0 = _proj(x, m1_W1, x.shape[1])

  p1 = _seg_sum(y1, u, v)
  y2, ge1 = _stage_mid(y1, p1, r(m1_b1), m1_W2, r(m1_b2), m2_W1)

  p2 = _seg_sum(y2, u, v)
  ne2, ge2 = _stage_last(y2, p2, r(m2_b1), m2_W2, r(m2_b2))

  return _final(ne2, ge0, ge1, ge2, g0_W, r(g0_b), g1_W, r(g1_b),
                g2_W, r(g2_b), gl_W, r(gl_b))


def kernel(x, edge_index, g0_W, g0_b, g1_W, g1_b, g2_W, g2_b, gl_W, gl_b,
           m1_W1, m1_b1, m1_W2, m1_b2, m2_W1, m2_b1, m2_W2, m2_b2):
  npad_e = _EPAD - _E
  if npad_e:
    u = jnp.concatenate([edge_index[0], jnp.zeros((npad_e,), jnp.int32)])
    v = jnp.concatenate(
        [edge_index[1], jnp.full((npad_e,), _NPAD - 1, jnp.int32)])
  else:
    u = edge_index[0]
    v = edge_index[1]
  return _run(x, u, v, g0_W, g0_b, g1_W, g1_b, g2_W, g2_b, gl_W, gl_b,
              m1_W1, m1_b1, m1_W2, m1_b2, m2_W1, m2_b1, m2_W2, m2_b2)


# speedup vs baseline: 3.3686x; 3.1860x over previous
import functools

import jax
import jax.numpy as jnp
from jax import lax
from jax.experimental import pallas as pl
from jax.experimental.pallas import tpu as pltpu
from jax.experimental.pallas import tpu_sc as plsc

_N = 10000
_D = 128
_E = 320000

_NUM_CORES = 2
_NUM_SUBCORES = 16
_NW = _NUM_CORES * _NUM_SUBCORES
_CHUNK = 80
_NCHUNK = 128
_EPAD = _NW * _NCHUNK * _CHUNK
_NPAD = 10240
_ROWS_PER_TILE = _NPAD // _NUM_SUBCORES
_ZROWS = 128

_ROW_BLOCK = 1000
_GRID = _N // _ROW_BLOCK


def _make_seg_sum():
  mesh = plsc.VectorSubcoreMesh(core_axis_name="c", subcore_axis_name="s")

  @functools.partial(
      pl.kernel,
      mesh=mesh,
      out_type=jax.ShapeDtypeStruct((_NUM_CORES, _NPAD, _D), jnp.float32),
      scratch_types=[
          pltpu.VMEM((_CHUNK,), jnp.int32),
          pltpu.VMEM((_CHUNK,), jnp.int32),
          pltpu.VMEM((_CHUNK,), jnp.int32),
          pltpu.VMEM((_CHUNK,), jnp.int32),
          pltpu.VMEM((_CHUNK, _D), jnp.float32),
          pltpu.VMEM((_CHUNK, _D), jnp.float32),
          pltpu.VMEM((_ZROWS, _D), jnp.float32),
          pltpu.VMEM_SHARED((_NPAD, _D), jnp.float32),
          pltpu.SemaphoreType.DMA,
          pltpu.SemaphoreType.DMA,
          pltpu.SemaphoreType.DMA,
          pltpu.SemaphoreType.DMA,
      ],
  )
  def seg_sum(table_hbm, u_hbm, v_hbm, out_hbm, u0_v, u1_v, v0_v, v1_v,
              rows0_v, rows1_v, stage_v, acc_sh, isem0, isem1, rsem0, rsem1):
    c = lax.axis_index("c")
    s = lax.axis_index("s")
    wid = s * _NUM_CORES + c
    base0 = wid * _NCHUNK * _CHUNK

    uu = (u0_v, u1_v)
    vv = (v0_v, v1_v)
    isem = (isem0, isem1)
    rows = (rows0_v, rows1_v)
    rsem = (rsem0, rsem1)

    def issue_idx(g, b):
      off = pl.multiple_of(base0 + g * _CHUNK, 8)
      pltpu.async_copy(u_hbm.at[pl.ds(off, _CHUNK)], uu[b], isem[b])
      pltpu.async_copy(v_hbm.at[pl.ds(off, _CHUNK)], vv[b], isem[b])

    def wait_idx(b):
      pltpu.make_async_copy(u_hbm.at[pl.ds(0, _CHUNK)], uu[b],
                            isem[b]).wait()
      pltpu.make_async_copy(v_hbm.at[pl.ds(0, _CHUNK)], vv[b],
                            isem[b]).wait()

    issue_idx(0, 0)
    issue_idx(1, 1)

    def zero_row(r, _):
      def zero_col(cc, _):
        stage_v[r, pl.ds(cc * 16, 16)] = jnp.zeros((16,), jnp.float32)
        return 0
      return lax.fori_loop(0, _D // 16, zero_col, 0)
    lax.fori_loop(0, _ZROWS, zero_row, 0)

    row0 = s * _ROWS_PER_TILE
    def zcopy(k, _):
      pltpu.sync_copy(stage_v, acc_sh.at[pl.ds(row0 + k * _ZROWS, _ZROWS)])
      return 0
    lax.fori_loop(0, _ROWS_PER_TILE // _ZROWS, zcopy, 0)

    wait_idx(0)
    plsc.subcore_barrier()

    pltpu.async_copy(table_hbm.at[u0_v], rows0_v, rsem0)

    def step(g, b):
      pltpu.make_async_copy(table_hbm.at[uu[b]], rows[b], rsem[b]).wait()
      @pl.when(g + 1 < _NCHUNK)
      def _():
        wait_idx(1 - b)
        pltpu.async_copy(table_hbm.at[uu[1 - b]], rows[1 - b], rsem[1 - b])
      pltpu.sync_copy(rows[b], acc_sh.at[vv[b]], add=True)
      @pl.when(g + 2 < _NCHUNK)
      def _():
        issue_idx(g + 2, b)

    def body(i, _):
      for b in range(2):
        step(i * 2 + b, b)
      return 0
    lax.fori_loop(0, _NCHUNK // 2, body, 0)

    plsc.subcore_barrier()

    def wb(k, _):
      sl = pl.ds(row0 + k * _ZROWS, _ZROWS)
      pltpu.sync_copy(acc_sh.at[sl], stage_v)
      pltpu.sync_copy(stage_v, out_hbm.at[c].at[sl])
      return 0
    lax.fori_loop(0, _ROWS_PER_TILE // _ZROWS, wb, 0)

  return seg_sum


_seg_sum = _make_seg_sum()


def _dotT(a, w):
  return lax.dot_general(a, w, (((1,), (1,)), ((), ())),
                         preferred_element_type=jnp.float32)


def _proj_body(x_ref, w_ref, y_ref, ge_ref):
  x = x_ref[...]
  y_ref[...] = _dotT(x, w_ref[...])
  @pl.when(pl.program_id(0) == 0)
  def _():
    ge_ref[...] = jnp.zeros_like(ge_ref)
  ge_ref[...] += jnp.sum(x, axis=0, keepdims=True)


def _stage_mid_body(y_ref, p0_ref, p1_ref, b1_ref, w2_ref, b2_ref, wn_ref,
                    ynext_ref, ge_ref):
  z = jnp.maximum(y_ref[...] + p0_ref[0] + p1_ref[0] + b1_ref[...], 0.0)
  ne = jnp.maximum(_dotT(z, w2_ref[...]) + b2_ref[...], 0.0)
  ynext_ref[...] = _dotT(ne, wn_ref[...])
  @pl.when(pl.program_id(0) == 0)
  def _():
    ge_ref[...] = jnp.zeros_like(ge_ref)
  ge_ref[...] += jnp.sum(ne, axis=0, keepdims=True)


def _stage_last_body(y_ref, p0_ref, p1_ref, b1_ref, w2_ref, b2_ref,
                     ne_ref, ge_ref):
  z = jnp.maximum(y_ref[...] + p0_ref[0] + p1_ref[0] + b1_ref[...], 0.0)
  ne = jnp.maximum(_dotT(z, w2_ref[...]) + b2_ref[...], 0.0)
  ne_ref[...] = ne
  @pl.when(pl.program_id(0) == 0)
  def _():
    ge_ref[...] = jnp.zeros_like(ge_ref)
  ge_ref[...] += jnp.sum(ne, axis=0, keepdims=True)


def _final_body(ne_ref, ge0_ref, ge1_ref, ge2_ref, g0w_ref, g0b_ref,
                g1w_ref, g1b_ref, g2w_ref, g2b_ref, glw_ref, glb_ref,
                out_ref):
  o0 = jnp.maximum(_dotT(ge0_ref[...], g0w_ref[...]) + g0b_ref[...], 0.0)
  o1 = jnp.maximum(_dotT(ge1_ref[...], g1w_ref[...]) + g1b_ref[...], 0.0)
  o2 = jnp.maximum(_dotT(ge2_ref[...], g2w_ref[...]) + g2b_ref[...], 0.0)
  s = o0 + o1 + o2
  olast = jnp.maximum(_dotT(s, glw_ref[...]) + glb_ref[...], 0.0)
  out_ref[...] = ne_ref[...] + olast


def _row_spec(d):
  return pl.BlockSpec((_ROW_BLOCK, d), lambda i: (i, 0))


def _full_spec(shape):
  return pl.BlockSpec(shape, lambda i: tuple(0 for _ in shape))


def _proj(x, w, d_in):
  return pl.pallas_call(
      _proj_body,
      grid=(_GRID,),
      in_specs=[_row_spec(d_in), _full_spec(w.shape)],
      out_specs=[_row_spec(_D), _full_spec((1, d_in))],
      out_shape=[jax.ShapeDtypeStruct((_N, _D), jnp.float32),
                 jax.ShapeDtypeStruct((1, d_in), jnp.float32)],
  )(x, w)


def _stage_mid(y, p, b1, w2, b2, wn):
  return pl.pallas_call(
      _stage_mid_body,
      grid=(_GRID,),
      in_specs=[_row_spec(_D),
                pl.BlockSpec((1, _ROW_BLOCK, _D), lambda i: (0, i, 0)),
                pl.BlockSpec((1, _ROW_BLOCK, _D), lambda i: (1, i, 0)),
                _full_spec((1, _D)), _full_spec((_D, _D)),
                _full_spec((1, _D)), _full_spec((_D, _D))],
      out_specs=[_row_spec(_D), _full_spec((1, _D))],
      out_shape=[jax.ShapeDtypeStruct((_N, _D), jnp.float32),
                 jax.ShapeDtypeStruct((1, _D), jnp.float32)],
  )(y, p, p, b1, w2, b2, wn)


def _stage_last(y, p, b1, w2, b2):
  return pl.pallas_call(
      _stage_last_body,
      grid=(_GRID,),
      in_specs=[_row_spec(_D),
                pl.BlockSpec((1, _ROW_BLOCK, _D), lambda i: (0, i, 0)),
                pl.BlockSpec((1, _ROW_BLOCK, _D), lambda i: (1, i, 0)),
                _full_spec((1, _D)), _full_spec((_D, _D)),
                _full_spec((1, _D))],
      out_specs=[_row_spec(_D), _full_spec((1, _D))],
      out_shape=[jax.ShapeDtypeStruct((_N, _D), jnp.float32),
                 jax.ShapeDtypeStruct((1, _D), jnp.float32)],
  )(y, p, p, b1, w2, b2)


def _final(ne2, ge0, ge1, ge2, g0w, g0b, g1w, g1b, g2w, g2b, glw, glb):
  d_in = ge0.shape[1]
  return pl.pallas_call(
      _final_body,
      grid=(_GRID,),
      in_specs=[_row_spec(_D),
                _full_spec((1, d_in)), _full_spec((1, _D)), _full_spec((1, _D)),
                _full_spec((_D, d_in)), _full_spec((1, _D)),
                _full_spec((_D, _D)), _full_spec((1, _D)),
                _full_spec((_D, _D)), _full_spec((1, _D)),
                _full_spec((_D, _D)), _full_spec((1, _D))],
      out_specs=_row_spec(_D),
      out_shape=jax.ShapeDtypeStruct((_N, _D), jnp.float32),
  )(ne2, ge0, ge1, ge2, g0w, g0b, g1w, g1b, g2w, g2b, glw, glb)


@jax.jit
def _run(x, u, v, g0_W, g0_b, g1_W, g1_b, g2_W, g2_b, gl_W, gl_b,
         m1_W1, m1_b1, m1_W2, m1_b2, m2_W1, m2_b1, m2_W2, m2_b2):
  r = lambda b: b.reshape(1, -1)

  y1, ge0 = _proj(x, m1_W1, x.shape[1])

  p1 = _seg_sum(y1, u, v)
  y2, ge1 = _stage_mid(y1, p1, r(m1_b1), m1_W2, r(m1_b2), m2_W1)

  p2 = _seg_sum(y2, u, v)
  ne2, ge2 = _stage_last(y2, p2, r(m2_b1), m2_W2, r(m2_b2))

  return _final(ne2, ge0, ge1, ge2, g0_W, r(g0_b), g1_W, r(g1_b),
                g2_W, r(g2_b), gl_W, r(gl_b))


def kernel(x, edge_index, g0_W, g0_b, g1_W, g1_b, g2_W, g2_b, gl_W, gl_b,
           m1_W1, m1_b1, m1_W2, m1_b2, m2_W1, m2_b1, m2_W2, m2_b2):
  npad_e = _EPAD - _E
  if npad_e:
    pad_i = jnp.arange(npad_e, dtype=jnp.int32)
    u = jnp.concatenate([edge_index[0], pad_i % _N])
    v = jnp.concatenate([edge_index[1], _N + pad_i % (_NPAD - _N)])
  else:
    u = edge_index[0]
    v = edge_index[1]
  return _run(x, u, v, g0_W, g0_b, g1_W, g1_b, g2_W, g2_b, gl_W, gl_b,
              m1_W1, m1_b1, m1_W2, m1_b2, m2_W1, m2_b1, m2_W2, m2_b2)


# speedup vs baseline: 3.9050x; 1.1592x over previous
import functools

import jax
import jax.numpy as jnp
from jax import lax
from jax.experimental import pallas as pl
from jax.experimental.pallas import tpu as pltpu
from jax.experimental.pallas import tpu_sc as plsc

_N = 10000
_D = 128
_E = 320000

_NUM_CORES = 2
_NUM_SUBCORES = 16
_NW = _NUM_CORES * _NUM_SUBCORES
_CHUNK = 128
_NCHUNK = 80
_EPAD = _NW * _NCHUNK * _CHUNK
_NPAD = 10240
_ROWS_PER_TILE = _NPAD // _NUM_SUBCORES
_ZROWS = 64

_ROW_BLOCK = 1000
_GRID = _N // _ROW_BLOCK


def _make_seg_sum():
  mesh = plsc.VectorSubcoreMesh(core_axis_name="c", subcore_axis_name="s")

  @functools.partial(
      pl.kernel,
      mesh=mesh,
      out_type=jax.ShapeDtypeStruct((_NUM_CORES, _NPAD, _D), jnp.float32),
      scratch_types=[
          pltpu.VMEM((_CHUNK,), jnp.int32),
          pltpu.VMEM((_CHUNK,), jnp.int32),
          pltpu.VMEM((_CHUNK,), jnp.int32),
          pltpu.VMEM((_CHUNK,), jnp.int32),
          pltpu.VMEM((_CHUNK, _D), jnp.float32),
          pltpu.VMEM((_CHUNK, _D), jnp.float32),
          pltpu.VMEM((_ZROWS, _D), jnp.float32),
          pltpu.VMEM_SHARED((_NPAD, _D), jnp.float32),
          pltpu.SemaphoreType.DMA,
          pltpu.SemaphoreType.DMA,
          pltpu.SemaphoreType.DMA,
          pltpu.SemaphoreType.DMA,
      ],
  )
  def seg_sum(table_hbm, u_hbm, v_hbm, out_hbm, u0_v, u1_v, v0_v, v1_v,
              rows0_v, rows1_v, stage_v, acc_sh, isem0, isem1, rsem0, rsem1):
    c = lax.axis_index("c")
    s = lax.axis_index("s")
    wid = s * _NUM_CORES + c
    base0 = wid * _NCHUNK * _CHUNK

    uu = (u0_v, u1_v)
    vv = (v0_v, v1_v)
    isem = (isem0, isem1)
    rows = (rows0_v, rows1_v)
    rsem = (rsem0, rsem1)

    def issue_idx(g, b):
      off = pl.multiple_of(base0 + g * _CHUNK, 8)
      pltpu.async_copy(u_hbm.at[pl.ds(off, _CHUNK)], uu[b], isem[b])
      pltpu.async_copy(v_hbm.at[pl.ds(off, _CHUNK)], vv[b], isem[b])

    def wait_idx(b):
      pltpu.make_async_copy(u_hbm.at[pl.ds(0, _CHUNK)], uu[b],
                            isem[b]).wait()
      pltpu.make_async_copy(v_hbm.at[pl.ds(0, _CHUNK)], vv[b],
                            isem[b]).wait()

    issue_idx(0, 0)
    issue_idx(1, 1)

    def zero_row(r, _):
      def zero_col(cc, _):
        stage_v[r, pl.ds(cc * 16, 16)] = jnp.zeros((16,), jnp.float32)
        return 0
      return lax.fori_loop(0, _D // 16, zero_col, 0)
    lax.fori_loop(0, _ZROWS, zero_row, 0)

    row0 = s * _ROWS_PER_TILE
    def zcopy(k, _):
      pltpu.sync_copy(stage_v, acc_sh.at[pl.ds(row0 + k * _ZROWS, _ZROWS)])
      return 0
    lax.fori_loop(0, _ROWS_PER_TILE // _ZROWS, zcopy, 0)

    wait_idx(0)
    plsc.subcore_barrier()

    pltpu.async_copy(table_hbm.at[u0_v], rows0_v, rsem0)

    def step(g, b):
      pltpu.make_async_copy(table_hbm.at[uu[b]], rows[b], rsem[b]).wait()
      @pl.when(g + 1 < _NCHUNK)
      def _():
        wait_idx(1 - b)
        pltpu.async_copy(table_hbm.at[uu[1 - b]], rows[1 - b], rsem[1 - b])
      pltpu.sync_copy(rows[b], acc_sh.at[vv[b]], add=True)
      @pl.when(g + 2 < _NCHUNK)
      def _():
        issue_idx(g + 2, b)

    def body(i, _):
      for b in range(2):
        step(i * 2 + b, b)
      return 0
    lax.fori_loop(0, _NCHUNK // 2, body, 0)

    plsc.subcore_barrier()

    def wb(k, _):
      sl = pl.ds(row0 + k * _ZROWS, _ZROWS)
      pltpu.sync_copy(acc_sh.at[sl], stage_v)
      pltpu.sync_copy(stage_v, out_hbm.at[c].at[sl])
      return 0
    lax.fori_loop(0, _ROWS_PER_TILE // _ZROWS, wb, 0)

  return seg_sum


_seg_sum = _make_seg_sum()


def _dotT(a, w):
  return lax.dot_general(a, w, (((1,), (1,)), ((), ())),
                         preferred_element_type=jnp.float32)


def _proj_body(x_ref, w_ref, y_ref, ge_ref):
  x = x_ref[...]
  y_ref[...] = _dotT(x, w_ref[...])
  @pl.when(pl.program_id(0) == 0)
  def _():
    ge_ref[...] = jnp.zeros_like(ge_ref)
  ge_ref[...] += jnp.sum(x, axis=0, keepdims=True)


def _stage_mid_body(y_ref, p0_ref, p1_ref, b1_ref, w2_ref, b2_ref, wn_ref,
                    ynext_ref, ge_ref):
  z = jnp.maximum(y_ref[...] + p0_ref[0] + p1_ref[0] + b1_ref[...], 0.0)
  ne = jnp.maximum(_dotT(z, w2_ref[...]) + b2_ref[...], 0.0)
  ynext_ref[...] = _dotT(ne, wn_ref[...])
  @pl.when(pl.program_id(0) == 0)
  def _():
    ge_ref[...] = jnp.zeros_like(ge_ref)
  ge_ref[...] += jnp.sum(ne, axis=0, keepdims=True)


def _stage_last_body(y_ref, p0_ref, p1_ref, b1_ref, w2_ref, b2_ref,
                     ne_ref, ge_ref):
  z = jnp.maximum(y_ref[...] + p0_ref[0] + p1_ref[0] + b1_ref[...], 0.0)
  ne = jnp.maximum(_dotT(z, w2_ref[...]) + b2_ref[...], 0.0)
  ne_ref[...] = ne
  @pl.when(pl.program_id(0) == 0)
  def _():
    ge_ref[...] = jnp.zeros_like(ge_ref)
  ge_ref[...] += jnp.sum(ne, axis=0, keepdims=True)


def _final_body(ne_ref, ge0_ref, ge1_ref, ge2_ref, g0w_ref, g0b_ref,
                g1w_ref, g1b_ref, g2w_ref, g2b_ref, glw_ref, glb_ref,
                out_ref):
  o0 = jnp.maximum(_dotT(ge0_ref[...], g0w_ref[...]) + g0b_ref[...], 0.0)
  o1 = jnp.maximum(_dotT(ge1_ref[...], g1w_ref[...]) + g1b_ref[...], 0.0)
  o2 = jnp.maximum(_dotT(ge2_ref[...], g2w_ref[...]) + g2b_ref[...], 0.0)
  s = o0 + o1 + o2
  olast = jnp.maximum(_dotT(s, glw_ref[...]) + glb_ref[...], 0.0)
  out_ref[...] = ne_ref[...] + olast


def _row_spec(d):
  return pl.BlockSpec((_ROW_BLOCK, d), lambda i: (i, 0))


def _full_spec(shape):
  return pl.BlockSpec(shape, lambda i: tuple(0 for _ in shape))


def _proj(x, w, d_in):
  return pl.pallas_call(
      _proj_body,
      grid=(_GRID,),
      in_specs=[_row_spec(d_in), _full_spec(w.shape)],
      out_specs=[_row_spec(_D), _full_spec((1, d_in))],
      out_shape=[jax.ShapeDtypeStruct((_N, _D), jnp.float32),
                 jax.ShapeDtypeStruct((1, d_in), jnp.float32)],
  )(x, w)


def _stage_mid(y, p, b1, w2, b2, wn):
  return pl.pallas_call(
      _stage_mid_body,
      grid=(_GRID,),
      in_specs=[_row_spec(_D),
                pl.BlockSpec((1, _ROW_BLOCK, _D), lambda i: (0, i, 0)),
                pl.BlockSpec((1, _ROW_BLOCK, _D), lambda i: (1, i, 0)),
                _full_spec((1, _D)), _full_spec((_D, _D)),
                _full_spec((1, _D)), _full_spec((_D, _D))],
      out_specs=[_row_spec(_D), _full_spec((1, _D))],
      out_shape=[jax.ShapeDtypeStruct((_N, _D), jnp.float32),
                 jax.ShapeDtypeStruct((1, _D), jnp.float32)],
  )(y, p, p, b1, w2, b2, wn)


def _stage_last(y, p, b1, w2, b2):
  return pl.pallas_call(
      _stage_last_body,
      grid=(_GRID,),
      in_specs=[_row_spec(_D),
                pl.BlockSpec((1, _ROW_BLOCK, _D), lambda i: (0, i, 0)),
                pl.BlockSpec((1, _ROW_BLOCK, _D), lambda i: (1, i, 0)),
                _full_spec((1, _D)), _full_spec((_D, _D)),
                _full_spec((1, _D))],
      out_specs=[_row_spec(_D), _full_spec((1, _D))],
      out_shape=[jax.ShapeDtypeStruct((_N, _D), jnp.float32),
                 jax.ShapeDtypeStruct((1, _D), jnp.float32)],
  )(y, p, p, b1, w2, b2)


def _final(ne2, ge0, ge1, ge2, g0w, g0b, g1w, g1b, g2w, g2b, glw, glb):
  d_in = ge0.shape[1]
  return pl.pallas_call(
      _final_body,
      grid=(_GRID,),
      in_specs=[_row_spec(_D),
                _full_spec((1, d_in)), _full_spec((1, _D)), _full_spec((1, _D)),
                _full_spec((_D, d_in)), _full_spec((1, _D)),
                _full_spec((_D, _D)), _full_spec((1, _D)),
                _full_spec((_D, _D)), _full_spec((1, _D)),
                _full_spec((_D, _D)), _full_spec((1, _D))],
      out_specs=_row_spec(_D),
      out_shape=jax.ShapeDtypeStruct((_N, _D), jnp.float32),
  )(ne2, ge0, ge1, ge2, g0w, g0b, g1w, g1b, g2w, g2b, glw, glb)


@jax.jit
def _run(x, u, v, g0_W, g0_b, g1_W, g1_b, g2_W, g2_b, gl_W, gl_b,
         m1_W1, m1_b1, m1_W2, m1_b2, m2_W1, m2_b1, m2_W2, m2_b2):
  r = lambda b: b.reshape(1, -1)

  y1, ge0 = _proj(x, m1_W1, x.shape[1])

  p1 = _seg_sum(y1, u, v)
  y2, ge1 = _stage_mid(y1, p1, r(m1_b1), m1_W2, r(m1_b2), m2_W1)

  p2 = _seg_sum(y2, u, v)
  ne2, ge2 = _stage_last(y2, p2, r(m2_b1), m2_W2, r(m2_b2))

  return _final(ne2, ge0, ge1, ge2, g0_W, r(g0_b), g1_W, r(g1_b),
                g2_W, r(g2_b), gl_W, r(gl_b))


def kernel(x, edge_index, g0_W, g0_b, g1_W, g1_b, g2_W, g2_b, gl_W, gl_b,
           m1_W1, m1_b1, m1_W2, m1_b2, m2_W1, m2_b1, m2_W2, m2_b2):
  npad_e = _EPAD - _E
  if npad_e:
    pad_i = jnp.arange(npad_e, dtype=jnp.int32)
    u = jnp.concatenate([edge_index[0], pad_i % _N])
    v = jnp.concatenate([edge_index[1], _N + pad_i % (_NPAD - _N)])
  else:
    u = edge_index[0]
    v = edge_index[1]
  return _run(x, u, v, g0_W, g0_b, g1_W, g1_b, g2_W, g2_b, gl_W, gl_b,
              m1_W1, m1_b1, m1_W2, m1_b2, m2_W1, m2_b1, m2_W2, m2_b2)
